# Initial kernel scaffold; baseline (speedup 1.0000x reference)
import jax
import jax.numpy as jnp
from jax import lax
from jax.experimental import pallas as pl
from jax.experimental.pallas import tpu as pltpu
from jax.experimental.pallas import tpu_sc as plsc

N = 10000
E = 320000
D = 128

NC = 2
NS = 16
NW = NC * NS
CHUNK = 128
NCHUNKS = E // CHUNK
ROWS_PER_TILE = N // NS


def _sc_scatter_body(hw_hbm, src_hbm, dst_hbm, ew_hbm, out_hbm,
                     src_v, dst_v, ew_v, rows_v, acc, gsem):
    c = lax.axis_index("c")
    s = lax.axis_index("s")
    wid = s * NC + c

    zv = jnp.zeros((16,), jnp.float32)

    @pl.loop(0, CHUNK)
    def _zero(r):
        for j in range(D // 16):
            rows_v[r, pl.ds(j * 16, 16)] = zv

    base_row = s * ROWS_PER_TILE
    for t in range(ROWS_PER_TILE // CHUNK):
        pltpu.sync_copy(rows_v.at[:],
                        acc.at[pl.ds(base_row + t * CHUNK, CHUNK)])
    rem = ROWS_PER_TILE % CHUNK
    if rem:
        pltpu.sync_copy(rows_v.at[pl.ds(0, rem)],
                        acc.at[pl.ds(base_row + (ROWS_PER_TILE // CHUNK) * CHUNK,
                                     rem)])
    plsc.subcore_barrier()

    extra = NCHUNKS - (NCHUNKS // NW) * NW
    n_chunks = (NCHUNKS // NW) + jnp.where(wid < extra, 1, 0)

    def chunk_body(k, carry):
        base = (wid + k * NW) * CHUNK
        pltpu.sync_copy(src_hbm.at[pl.ds(base, CHUNK)], src_v)
        pltpu.sync_copy(ew_hbm.at[pl.ds(base, CHUNK)], ew_v)
        pltpu.sync_copy(dst_hbm.at[pl.ds(base, CHUNK)], dst_v)
        pltpu.async_copy(hw_hbm.at[src_v], rows_v, gsem).wait()

        def scale_body(e, c2):
            idx16 = jnp.zeros((16,), jnp.int32) + e
            wv = plsc.load_gather(ew_v, [idx16])
            for j in range(D // 16):
                sl = rows_v[e, pl.ds(j * 16, 16)]
                rows_v[e, pl.ds(j * 16, 16)] = sl * wv
            return c2

        lax.fori_loop(0, CHUNK, scale_body, 0)
        pltpu.sync_copy(rows_v, acc.at[dst_v], add=True)
        return carry

    lax.fori_loop(0, n_chunks, chunk_body, 0)
    plsc.subcore_barrier()

    pltpu.sync_copy(acc.at[pl.ds(base_row, ROWS_PER_TILE)],
                    out_hbm.at[c, pl.ds(base_row, ROWS_PER_TILE)])


_sc_scatter = pl.kernel(
    _sc_scatter_body,
    out_type=jax.ShapeDtypeStruct((NC, N, D), jnp.float32),
    mesh=plsc.VectorSubcoreMesh(core_axis_name="c", subcore_axis_name="s"),
    scratch_types=[
        pltpu.VMEM((CHUNK,), jnp.int32),
        pltpu.VMEM((CHUNK,), jnp.int32),
        pltpu.VMEM((CHUNK,), jnp.float32),
        pltpu.VMEM((CHUNK, D), jnp.float32),
        pltpu.VMEM_SHARED((N, D), jnp.float32),
        pltpu.SemaphoreType.DMA,
    ],
)


def _mm_body(x_ref, w_ref, o_ref):
    o_ref[...] = jnp.dot(x_ref[...], w_ref[...],
                         preferred_element_type=jnp.float32)


def _fuse_body(p_ref, b_ref, w_ref, o_ref):
    h = jnp.maximum(p_ref[0] + p_ref[1] + b_ref[...], 0.0)
    o_ref[...] = jnp.dot(h, w_ref[...], preferred_element_type=jnp.float32)


def _final_body(p_ref, b_ref, o_ref):
    o_ref[...] = p_ref[0] + p_ref[1] + b_ref[...]


@jax.jit
def kernel(x, edge_index, edge_weight, W1, b1, W2, b2):
    src = edge_index[0]
    dst = edge_index[1]
    b1r = b1.reshape(1, D)
    b2r = b2.reshape(1, D)

    hw1 = pl.pallas_call(
        _mm_body,
        out_shape=jax.ShapeDtypeStruct((N, D), jnp.float32),
    )(x, W1)

    p1 = _sc_scatter(hw1, src, dst, edge_weight)

    hw2 = pl.pallas_call(
        _fuse_body,
        out_shape=jax.ShapeDtypeStruct((N, D), jnp.float32),
    )(p1, b1r, W2)

    p2 = _sc_scatter(hw2, src, dst, edge_weight)

    out = pl.pallas_call(
        _final_body,
        out_shape=jax.ShapeDtypeStruct((N, D), jnp.float32),
    )(p2, b2r)
    return out


# speedup vs baseline: 5.2017x; 5.2017x over previous
import jax
import jax.numpy as jnp
from jax import lax
from jax.experimental import pallas as pl
from jax.experimental.pallas import tpu as pltpu
from jax.experimental.pallas import tpu_sc as plsc

N = 10000
E = 320000
D = 128

NC = 2
NS = 16
NW = NC * NS
CHUNK = 128
NCHUNKS = E // CHUNK
ROWS_PER_TILE = 624


def _sc_scatter_body(hw_hbm, src_hbm, dst_hbm, ew_hbm, out_hbm,
                     src_v, dst_v, ew_v, rows_v, acc, gsem):
    c = lax.axis_index("c")
    s = lax.axis_index("s")
    wid = s * NC + c

    zv = jnp.zeros((16,), jnp.float32)

    @pl.loop(0, CHUNK)
    def _zero(r):
        for j in range(D // 16):
            rows_v[r, pl.ds(j * 16, 16)] = zv

    base_row = s * ROWS_PER_TILE
    for t in range(ROWS_PER_TILE // CHUNK):
        pltpu.sync_copy(rows_v.at[:],
                        acc.at[pl.ds(base_row + t * CHUNK, CHUNK)])
    rem = ROWS_PER_TILE % CHUNK
    if rem:
        pltpu.sync_copy(rows_v.at[pl.ds(0, rem)],
                        acc.at[pl.ds(base_row + (ROWS_PER_TILE // CHUNK) * CHUNK,
                                     rem)])

    tail_base = NS * ROWS_PER_TILE
    @pl.when(s == NS - 1)
    def _zero_tail():
        pltpu.sync_copy(rows_v.at[pl.ds(0, N - tail_base)],
                        acc.at[pl.ds(tail_base, N - tail_base)])
    plsc.subcore_barrier()

    extra = NCHUNKS - (NCHUNKS // NW) * NW
    n_chunks = (NCHUNKS // NW) + jnp.where(wid < extra, 1, 0)

    def chunk_body(k, carry):
        base = (wid + k * NW) * CHUNK
        pltpu.sync_copy(src_hbm.at[pl.ds(base, CHUNK)], src_v)
        pltpu.sync_copy(ew_hbm.at[pl.ds(base, CHUNK)], ew_v)
        pltpu.sync_copy(dst_hbm.at[pl.ds(base, CHUNK)], dst_v)
        pltpu.async_copy(hw_hbm.at[src_v], rows_v, gsem).wait()

        def scale_body(g, c2):
            ewv = ew_v[pl.ds(g * 16, 16)]
            for l in range(16):
                e = g * 16 + l
                wv = jnp.full((16,), ewv[l], dtype=jnp.float32)
                for j in range(D // 16):
                    sl = rows_v[e, pl.ds(j * 16, 16)]
                    rows_v[e, pl.ds(j * 16, 16)] = sl * wv
            return c2

        lax.fori_loop(0, CHUNK // 16, scale_body, 0)
        pltpu.sync_copy(rows_v, acc.at[dst_v], add=True)
        return carry

    lax.fori_loop(0, n_chunks, chunk_body, 0)
    plsc.subcore_barrier()

    pltpu.sync_copy(acc.at[pl.ds(base_row, ROWS_PER_TILE)],
                    out_hbm.at[c, pl.ds(base_row, ROWS_PER_TILE)])

    @pl.when(s == NS - 1)
    def _dump_tail():
        pltpu.sync_copy(acc.at[pl.ds(tail_base, N - tail_base)],
                        out_hbm.at[c, pl.ds(tail_base, N - tail_base)])


_sc_scatter = pl.kernel(
    _sc_scatter_body,
    out_type=jax.ShapeDtypeStruct((NC, N, D), jnp.float32),
    mesh=plsc.VectorSubcoreMesh(core_axis_name="c", subcore_axis_name="s"),
    scratch_types=[
        pltpu.VMEM((CHUNK,), jnp.int32),
        pltpu.VMEM((CHUNK,), jnp.int32),
        pltpu.VMEM((CHUNK,), jnp.float32),
        pltpu.VMEM((CHUNK, D), jnp.float32),
        pltpu.VMEM_SHARED((N, D), jnp.float32),
        pltpu.SemaphoreType.DMA,
    ],
)


def _mm_body(x_ref, w_ref, o_ref):
    o_ref[...] = jnp.dot(x_ref[...], w_ref[...],
                         preferred_element_type=jnp.float32)


def _fuse_body(p_ref, b_ref, w_ref, o_ref):
    h = jnp.maximum(p_ref[0] + p_ref[1] + b_ref[...], 0.0)
    o_ref[...] = jnp.dot(h, w_ref[...], preferred_element_type=jnp.float32)


def _final_body(p_ref, b_ref, o_ref):
    o_ref[...] = p_ref[0] + p_ref[1] + b_ref[...]


@jax.jit
def kernel(x, edge_index, edge_weight, W1, b1, W2, b2):
    src = edge_index[0]
    dst = edge_index[1]
    b1r = b1.reshape(1, D)
    b2r = b2.reshape(1, D)

    hw1 = pl.pallas_call(
        _mm_body,
        out_shape=jax.ShapeDtypeStruct((N, D), jnp.float32),
    )(x, W1)

    p1 = _sc_scatter(hw1, src, dst, edge_weight)

    hw2 = pl.pallas_call(
        _fuse_body,
        out_shape=jax.ShapeDtypeStruct((N, D), jnp.float32),
    )(p1, b1r, W2)

    p2 = _sc_scatter(hw2, src, dst, edge_weight)

    out = pl.pallas_call(
        _final_body,
        out_shape=jax.ShapeDtypeStruct((N, D), jnp.float32),
    )(p2, b2r)
    return out


# speedup vs baseline: 10.0598x; 1.9340x over previous
import jax
import jax.numpy as jnp
from jax import lax
from jax.experimental import pallas as pl
from jax.experimental.pallas import tpu as pltpu
from jax.experimental.pallas import tpu_sc as plsc

N = 10000
E = 320000
D = 128

NC = 2
NS = 16
NW = NC * NS
CHUNK = 128
NCHUNKS = E // CHUNK
ROWS_PER_TILE = 624


def _sc_scatter_body(hw_hbm, src_hbm, dst_hbm, ew_hbm, out_hbm,
                     src_v, dst_v, ew_v, rows_v, acc, isem, gsem):
    c = lax.axis_index("c")
    s = lax.axis_index("s")
    wid = s * NC + c

    zv = jnp.zeros((16,), jnp.float32)

    @pl.loop(0, CHUNK)
    def _zero(r):
        for j in range(D // 16):
            rows_v[0, r, pl.ds(j * 16, 16)] = zv

    base_row = s * ROWS_PER_TILE
    for t in range(ROWS_PER_TILE // CHUNK):
        pltpu.sync_copy(rows_v.at[0],
                        acc.at[pl.ds(base_row + t * CHUNK, CHUNK)])
    rem = ROWS_PER_TILE % CHUNK
    if rem:
        pltpu.sync_copy(rows_v.at[0, pl.ds(0, rem)],
                        acc.at[pl.ds(base_row + (ROWS_PER_TILE // CHUNK) * CHUNK,
                                     rem)])

    tail_base = NS * ROWS_PER_TILE
    @pl.when(s == NS - 1)
    def _zero_tail():
        pltpu.sync_copy(rows_v.at[0, pl.ds(0, N - tail_base)],
                        acc.at[pl.ds(tail_base, N - tail_base)])
    plsc.subcore_barrier()

    extra = NCHUNKS - (NCHUNKS // NW) * NW
    n_chunks = (NCHUNKS // NW) + jnp.where(wid < extra, 1, 0)

    def idx_load(k, b):
        base = (wid + k * NW) * CHUNK
        pltpu.async_copy(src_hbm.at[pl.ds(base, CHUNK)], src_v.at[b], isem)
        pltpu.async_copy(ew_hbm.at[pl.ds(base, CHUNK)], ew_v.at[b], isem)
        pltpu.async_copy(dst_hbm.at[pl.ds(base, CHUNK)], dst_v.at[b], isem)

    def idx_wait(b):
        pltpu.make_async_copy(src_hbm.at[pl.ds(0, CHUNK)], src_v.at[b], isem).wait()
        pltpu.make_async_copy(ew_hbm.at[pl.ds(0, CHUNK)], ew_v.at[b], isem).wait()
        pltpu.make_async_copy(dst_hbm.at[pl.ds(0, CHUNK)], dst_v.at[b], isem).wait()

    def gather_start(b):
        pltpu.async_copy(hw_hbm.at[src_v.at[b]], rows_v.at[b], gsem)

    def gather_wait(b):
        pltpu.make_async_copy(hw_hbm.at[src_v.at[b]], rows_v.at[b], gsem).wait()

    def scale(b):
        def scale_body(g, c2):
            ewv = ew_v[b, pl.ds(g * 16, 16)]
            for l in range(16):
                e = g * 16 + l
                wv = jnp.full((16,), ewv[l], dtype=jnp.float32)
                for j in range(D // 16):
                    sl = rows_v[b, e, pl.ds(j * 16, 16)]
                    rows_v[b, e, pl.ds(j * 16, 16)] = sl * wv
            return c2
        lax.fori_loop(0, CHUNK // 16, scale_body, 0)

    idx_load(0, 0)
    idx_load(1, 1)
    idx_wait(0)
    gather_start(0)

    n_pairs = (n_chunks + 1) // 2

    def pair_body(p, carry):
        for b in range(2):
            k = p * 2 + b

            @pl.when(k < n_chunks)
            def _step():
                gather_wait(b)

                @pl.when(k + 1 < n_chunks)
                def _next_gather():
                    idx_wait(1 - b)
                    gather_start(1 - b)

                scale(b)
                pltpu.sync_copy(rows_v.at[b], acc.at[dst_v.at[b]], add=True)

                @pl.when(k + 2 < n_chunks)
                def _prefetch_idx():
                    idx_load(k + 2, b)
        return carry

    lax.fori_loop(0, n_pairs, pair_body, 0)
    plsc.subcore_barrier()

    pltpu.sync_copy(acc.at[pl.ds(base_row, ROWS_PER_TILE)],
                    out_hbm.at[c, pl.ds(base_row, ROWS_PER_TILE)])

    @pl.when(s == NS - 1)
    def _dump_tail():
        pltpu.sync_copy(acc.at[pl.ds(tail_base, N - tail_base)],
                        out_hbm.at[c, pl.ds(tail_base, N - tail_base)])


_sc_scatter = pl.kernel(
    _sc_scatter_body,
    out_type=jax.ShapeDtypeStruct((NC, N, D), jnp.float32),
    mesh=plsc.VectorSubcoreMesh(core_axis_name="c", subcore_axis_name="s"),
    scratch_types=[
        pltpu.VMEM((2, CHUNK), jnp.int32),
        pltpu.VMEM((2, CHUNK), jnp.int32),
        pltpu.VMEM((2, CHUNK), jnp.float32),
        pltpu.VMEM((2, CHUNK, D), jnp.float32),
        pltpu.VMEM_SHARED((N, D), jnp.float32),
        pltpu.SemaphoreType.DMA,
        pltpu.SemaphoreType.DMA,
    ],
)


def _mm_body(x_ref, w_ref, o_ref):
    o_ref[...] = jnp.dot(x_ref[...], w_ref[...],
                         preferred_element_type=jnp.float32)


def _fuse_body(p_ref, b_ref, w_ref, o_ref):
    h = jnp.maximum(p_ref[0] + p_ref[1] + b_ref[...], 0.0)
    o_ref[...] = jnp.dot(h, w_ref[...], preferred_element_type=jnp.float32)


def _final_body(p_ref, b_ref, o_ref):
    o_ref[...] = p_ref[0] + p_ref[1] + b_ref[...]


@jax.jit
def kernel(x, edge_index, edge_weight, W1, b1, W2, b2):
    src = edge_index[0]
    dst = edge_index[1]
    b1r = b1.reshape(1, D)
    b2r = b2.reshape(1, D)

    hw1 = pl.pallas_call(
        _mm_body,
        out_shape=jax.ShapeDtypeStruct((N, D), jnp.float32),
    )(x, W1)

    p1 = _sc_scatter(hw1, src, dst, edge_weight)

    hw2 = pl.pallas_call(
        _fuse_body,
        out_shape=jax.ShapeDtypeStruct((N, D), jnp.float32),
    )(p1, b1r, W2)

    p2 = _sc_scatter(hw2, src, dst, edge_weight)

    out = pl.pallas_call(
        _final_body,
        out_shape=jax.ShapeDtypeStruct((N, D), jnp.float32),
    )(p2, b2r)
    return out
